# baseline (speedup 1.0000x reference)
import jax
import jax.numpy as jnp
from jax import lax
from jax.experimental import pallas as pl
from jax.experimental.pallas import tpu as pltpu
from jax.experimental.pallas import tpu_sc as plsc

N = 262144
C = 64
GRID = 64
TABLE_ROWS = GRID * GRID

_NC = 2
_NS = 16
_NW = _NC * _NS
_L = 16

_QB = 128
_NQB = N // _QB
_CPW = _NQB // _NW
_PER_W = N // _NW
_PSTR = 129


def _body(table_hbm, x_hbm, out_hbm, x_v, idx_v, w_v, rows0, rows1,
          rows2, rows3, pack0, pack1, gsem0, gsem1, gsem2, gsem3,
          osem0, osem1):
    wid = lax.axis_index("s") * _NC + lax.axis_index("c")
    qb0 = wid * _CPW

    pltpu.sync_copy(x_hbm.at[pl.ds(qb0, _CPW)], x_v)

    lanes = lax.iota(jnp.int32, _L)
    zeros = jnp.zeros((_L,), jnp.int32)
    cap = jnp.full((_L,), GRID - 1, jnp.int32)

    @plsc.parallel_loop(0, _CPW, unroll=2)
    def _comp(ci):
        for j in range(_QB // _L):
            x0 = x_v[ci, 0, pl.ds(j * _L, _L)]
            x1 = x_v[ci, 1, pl.ds(j * _L, _L)]
            low0 = x0.astype(jnp.int32)
            low1 = x1.astype(jnp.int32)
            w0 = (low0 + 1).astype(jnp.float32) - x0
            w1 = (low1 + 1).astype(jnp.float32) - x1
            q0 = ci * _QB + j * _L
            idx_v[pl.ds(q0, _L)] = jnp.minimum(low0, cap) * GRID + jnp.minimum(low1, cap)
            w_v[pl.ds(q0, _L)] = w0 * w1

    rows = (rows0, rows1, rows2, rows3)
    pack = (pack0, pack1)
    gsem = (gsem0, gsem1, gsem2, gsem3)
    osem = (osem0, osem1)
    _GDEPTH = 4

    def fire_gather(ci, p):
        pltpu.async_copy(
            table_hbm.at[idx_v.at[pl.ds(ci * _QB, _QB)]], rows[p], gsem[p])

    def wait_out(p2):
        pltpu.make_async_copy(
            pack[p2].at[:, :, :, pl.ds(0, _QB)],
            out_hbm.at[:, pl.ds(0, 1)], osem[p2]).wait()

    cdims = []
    for cb16 in range(C // _L):
        cvec = cb16 * _L + lanes
        cdims.append((cvec >> 3, zeros, cvec & 7))

    for pre in range(_GDEPTH - 1):
        fire_gather(pre, pre)

    def quad_body(cq, carry):
        for par in range(_GDEPTH):
            ci = _GDEPTH * cq + par
            p2 = par % 2

            @pl.when(ci + _GDEPTH - 1 < _CPW)
            def _next_gather():
                fire_gather(ci + _GDEPTH - 1, (par + _GDEPTH - 1) % _GDEPTH)

            with jax.named_scope("gwait"):
                pltpu.make_async_copy(
                    table_hbm.at[pl.ds(0, _QB)], rows[par], gsem[par]).wait()

            with jax.named_scope("owait"):
                @pl.when(ci >= 2)
                def _wait_out():
                    wait_out(p2)

            woff = ci * _QB
            rows_p = rows[par]
            pack_p = pack[p2]

            with jax.named_scope("scale"):
                @plsc.parallel_loop(0, _QB, step=_L, unroll=2)
                def _scale(b0):
                    wvec = w_v[pl.ds(woff + b0, _L)]
                    for k in range(_L):
                        b = b0 + k
                        wq = jnp.full((_L,), wvec[k])
                        qvec = zeros + b
                        for cb16 in range(C // _L):
                            seg = rows_p[b, pl.ds(cb16 * _L, _L)]
                            d0, d1, d2 = cdims[cb16]
                            plsc.store_scatter(
                                pack_p, [d0, d1, d2, qvec], seg * wq)

            pltpu.async_copy(
                pack[p2].at[:, :, :, pl.ds(0, _QB)],
                out_hbm.at[:, pl.ds(qb0 + ci, 1)], osem[p2])
        return carry

    lax.fori_loop(0, _CPW // _GDEPTH, quad_body, 0)

    wait_out(0)
    wait_out(1)


_kern = pl.kernel(
    _body,
    out_type=jax.ShapeDtypeStruct((C // 8, _NQB, 8, _QB), jnp.float32),
    mesh=plsc.VectorSubcoreMesh(core_axis_name="c", subcore_axis_name="s"),
    scratch_types=[
        pltpu.VMEM((_CPW, 2, _QB), jnp.float32),
        pltpu.VMEM((_PER_W,), jnp.int32),
        pltpu.VMEM((_PER_W,), jnp.float32),
        pltpu.VMEM((_QB, C), jnp.float32),
        pltpu.VMEM((_QB, C), jnp.float32),
        pltpu.VMEM((_QB, C), jnp.float32),
        pltpu.VMEM((_QB, C), jnp.float32),
        pltpu.VMEM((C // 8, 1, 8, _PSTR), jnp.float32),
        pltpu.VMEM((C // 8, 1, 8, _PSTR), jnp.float32),
        pltpu.SemaphoreType.DMA,
        pltpu.SemaphoreType.DMA,
        pltpu.SemaphoreType.DMA,
        pltpu.SemaphoreType.DMA,
        pltpu.SemaphoreType.DMA,
        pltpu.SemaphoreType.DMA,
    ],
    compiler_params=pltpu.CompilerParams(
        needs_layout_passes=False, use_tc_tiling_on_sc=False
    ),
)


def kernel(image, x):
    table = image.reshape(TABLE_ROWS, C)
    xp = x.reshape(_NQB, _QB, 2).transpose(0, 2, 1)
    out_l = _kern(table, xp)
    return out_l.transpose(1, 3, 0, 2).reshape(N, C)


# speedup vs baseline: 1.0618x; 1.0618x over previous
import jax
import jax.numpy as jnp
from jax import lax
from jax.experimental import pallas as pl
from jax.experimental.pallas import tpu as pltpu
from jax.experimental.pallas import tpu_sc as plsc

N = 262144
C = 64
GRID = 64
TABLE_ROWS = GRID * GRID

_NC = 2
_NS = 16
_NW = _NC * _NS
_L = 16

_QB = 128
_NQB = N // _QB
_CPW = _NQB // _NW
_PER_W = N // _NW
_PSTR = 129


def _body(table_hbm, x_hbm, out_hbm, x_v, idx_v, w_v, rows0, rows1,
          pack0, pack1, gsem0, gsem1, osem0, osem1):
    wid = lax.axis_index("s") * _NC + lax.axis_index("c")
    qb0 = wid * _CPW

    pltpu.sync_copy(x_hbm.at[pl.ds(qb0, _CPW)], x_v)

    lanes = lax.iota(jnp.int32, _L)
    zeros = jnp.zeros((_L,), jnp.int32)
    cap = jnp.full((_L,), GRID - 1, jnp.int32)

    @plsc.parallel_loop(0, _CPW, unroll=2)
    def _comp(ci):
        for j in range(_QB // _L):
            x0 = x_v[ci, 0, pl.ds(j * _L, _L)]
            x1 = x_v[ci, 1, pl.ds(j * _L, _L)]
            low0 = x0.astype(jnp.int32)
            low1 = x1.astype(jnp.int32)
            w0 = (low0 + 1).astype(jnp.float32) - x0
            w1 = (low1 + 1).astype(jnp.float32) - x1
            q0 = ci * _QB + j * _L
            idx_v[pl.ds(q0, _L)] = jnp.minimum(low0, cap) * GRID + jnp.minimum(low1, cap)
            w_v[pl.ds(q0, _L)] = w0 * w1

    rows = (rows0, rows1)
    pack = (pack0, pack1)
    gsem = (gsem0, gsem1)
    osem = (osem0, osem1)

    def fire_gather(ci, p):
        pltpu.async_copy(
            table_hbm.at[idx_v.at[pl.ds(ci * _QB, _QB)]], rows[p], gsem[p])

    cdims = []
    for cb16 in range(C // _L):
        cvec = cb16 * _L + lanes
        cdims.append((cvec >> 3, zeros, cvec & 7))

    fire_gather(0, 0)

    def pair_body(cp, carry):
        for par in (0, 1):
            ci = 2 * cp + par
            q = 1 - par

            @pl.when(ci >= 1)
            def _wait_out():
                pltpu.make_async_copy(
                    pack[q].at[:, :, :, pl.ds(0, _QB)],
                    out_hbm.at[:, pl.ds(0, 1)], osem[q]).wait()

            @pl.when(ci + 1 < _CPW)
            def _next_gather():
                fire_gather(ci + 1, q)

            pltpu.make_async_copy(
                table_hbm.at[pl.ds(0, _QB)], rows[par], gsem[par]).wait()

            woff = ci * _QB
            rows_p = rows[par]
            pack_p = pack[par]

            @plsc.parallel_loop(0, _QB, step=_L, unroll=2)
            def _scale(b0):
                wvec = w_v[pl.ds(woff + b0, _L)]
                for k in range(_L):
                    b = b0 + k
                    wq = jnp.full((_L,), wvec[k])
                    qvec = zeros + b
                    for cb16 in range(C // _L):
                        seg = rows_p[b, pl.ds(cb16 * _L, _L)]
                        d0, d1, d2 = cdims[cb16]
                        plsc.store_scatter(
                            pack_p, [d0, d1, d2, qvec], seg * wq)

            pltpu.async_copy(
                pack[par].at[:, :, :, pl.ds(0, _QB)],
                out_hbm.at[:, pl.ds(qb0 + ci, 1)], osem[par])
        return carry

    lax.fori_loop(0, _CPW // 2, pair_body, 0)

    pltpu.make_async_copy(
        pack1.at[:, :, :, pl.ds(0, _QB)],
        out_hbm.at[:, pl.ds(0, 1)], osem1).wait()


_kern = pl.kernel(
    _body,
    out_type=jax.ShapeDtypeStruct((C // 8, _NQB, 8, _QB), jnp.float32),
    mesh=plsc.VectorSubcoreMesh(core_axis_name="c", subcore_axis_name="s"),
    scratch_types=[
        pltpu.VMEM((_CPW, 2, _QB), jnp.float32),
        pltpu.VMEM((_PER_W,), jnp.int32),
        pltpu.VMEM((_PER_W,), jnp.float32),
        pltpu.VMEM((_QB, C), jnp.float32),
        pltpu.VMEM((_QB, C), jnp.float32),
        pltpu.VMEM((C // 8, 1, 8, _PSTR), jnp.float32),
        pltpu.VMEM((C // 8, 1, 8, _PSTR), jnp.float32),
        pltpu.SemaphoreType.DMA,
        pltpu.SemaphoreType.DMA,
        pltpu.SemaphoreType.DMA,
        pltpu.SemaphoreType.DMA,
    ],
    compiler_params=pltpu.CompilerParams(
        needs_layout_passes=False, use_tc_tiling_on_sc=False
    ),
)


def kernel(image, x):
    table = image.reshape(TABLE_ROWS, C)
    xp = x.reshape(_NQB, _QB, 2).transpose(0, 2, 1)
    out_l = _kern(table, xp)
    return out_l.transpose(1, 3, 0, 2).reshape(N, C)


# speedup vs baseline: 1.0717x; 1.0093x over previous
import jax
import jax.numpy as jnp
from jax import lax
from jax.experimental import pallas as pl
from jax.experimental.pallas import tpu as pltpu
from jax.experimental.pallas import tpu_sc as plsc

N = 262144
C = 64
GRID = 64
TABLE_ROWS = GRID * GRID

_NC = 2
_NS = 16
_NW = _NC * _NS
_L = 16

_QB = 128
_NQB = N // _QB
_CPW = _NQB // _NW
_PER_W = N // _NW
_PSTR = 129


def _body(table_hbm, x_hbm, out_hbm, x_v, idx_v, w_v, rows0, rows1,
          pack0, pack1, gsem0, gsem1, osem0, osem1):
    wid = lax.axis_index("s") * _NC + lax.axis_index("c")
    qb0 = wid * _CPW

    pltpu.sync_copy(x_hbm.at[pl.ds(qb0, _CPW)], x_v)

    lanes = lax.iota(jnp.int32, _L)
    zeros = jnp.zeros((_L,), jnp.int32)
    cap = jnp.full((_L,), GRID - 1, jnp.int32)

    @plsc.parallel_loop(0, _CPW, unroll=2)
    def _comp(ci):
        for j in range(_QB // _L):
            x0 = x_v[ci, 0, pl.ds(j * _L, _L)]
            x1 = x_v[ci, 1, pl.ds(j * _L, _L)]
            low0 = x0.astype(jnp.int32)
            low1 = x1.astype(jnp.int32)
            w0 = (low0 + 1).astype(jnp.float32) - x0
            w1 = (low1 + 1).astype(jnp.float32) - x1
            q0 = ci * _QB + j * _L
            idx_v[pl.ds(q0, _L)] = jnp.minimum(low0, cap) * GRID + jnp.minimum(low1, cap)
            w_v[pl.ds(q0, _L)] = w0 * w1

    rows = (rows0, rows1)
    pack = (pack0, pack1)
    gsem = (gsem0, gsem1)
    osem = (osem0, osem1)

    def fire_gather(ci, p):
        pltpu.async_copy(
            table_hbm.at[idx_v.at[pl.ds(ci * _QB, _QB)]], rows[p], gsem[p])

    cdims = []
    for cb16 in range(C // _L):
        cvec = cb16 * _L + lanes
        cdims.append((cvec >> 3, zeros, cvec & 7))

    fire_gather(0, 0)

    def pair_body(cp, carry):
        for par in (0, 1):
            ci = 2 * cp + par
            q = 1 - par

            @pl.when(ci >= 1)
            def _wait_out():
                pltpu.make_async_copy(
                    pack[q].at[:, :, :, pl.ds(0, _QB)],
                    out_hbm.at[:, pl.ds(0, 1)], osem[q]).wait()

            @pl.when(ci + 1 < _CPW)
            def _next_gather():
                fire_gather(ci + 1, q)

            pltpu.make_async_copy(
                table_hbm.at[pl.ds(0, _QB)], rows[par], gsem[par]).wait()

            woff = ci * _QB
            rows_p = rows[par]
            pack_p = pack[par]

            @plsc.parallel_loop(0, _QB, step=_L, unroll=1)
            def _scale(b0):
                wvec = w_v[pl.ds(woff + b0, _L)]
                for k in range(_L):
                    b = b0 + k
                    wq = jnp.full((_L,), wvec[k])
                    qvec = zeros + b
                    for cb16 in range(C // _L):
                        seg = rows_p[b, pl.ds(cb16 * _L, _L)]
                        d0, d1, d2 = cdims[cb16]
                        plsc.store_scatter(
                            pack_p, [d0, d1, d2, qvec], seg * wq)

            pltpu.async_copy(
                pack[par].at[:, :, :, pl.ds(0, _QB)],
                out_hbm.at[:, pl.ds(qb0 + ci, 1)], osem[par])
        return carry

    lax.fori_loop(0, _CPW // 2, pair_body, 0)

    pltpu.make_async_copy(
        pack1.at[:, :, :, pl.ds(0, _QB)],
        out_hbm.at[:, pl.ds(0, 1)], osem1).wait()


_kern = pl.kernel(
    _body,
    out_type=jax.ShapeDtypeStruct((C // 8, _NQB, 8, _QB), jnp.float32),
    mesh=plsc.VectorSubcoreMesh(core_axis_name="c", subcore_axis_name="s"),
    scratch_types=[
        pltpu.VMEM((_CPW, 2, _QB), jnp.float32),
        pltpu.VMEM((_PER_W,), jnp.int32),
        pltpu.VMEM((_PER_W,), jnp.float32),
        pltpu.VMEM((_QB, C), jnp.float32),
        pltpu.VMEM((_QB, C), jnp.float32),
        pltpu.VMEM((C // 8, 1, 8, _PSTR), jnp.float32),
        pltpu.VMEM((C // 8, 1, 8, _PSTR), jnp.float32),
        pltpu.SemaphoreType.DMA,
        pltpu.SemaphoreType.DMA,
        pltpu.SemaphoreType.DMA,
        pltpu.SemaphoreType.DMA,
    ],
    compiler_params=pltpu.CompilerParams(
        needs_layout_passes=False, use_tc_tiling_on_sc=False
    ),
)


def kernel(image, x):
    table = image.reshape(TABLE_ROWS, C)
    xp = x.reshape(_NQB, _QB, 2).transpose(0, 2, 1)
    out_l = _kern(table, xp)
    return out_l.transpose(1, 3, 0, 2).reshape(N, C)


# speedup vs baseline: 1.4650x; 1.3670x over previous
import jax
import jax.numpy as jnp
from jax import lax
from jax.experimental import pallas as pl
from jax.experimental.pallas import tpu as pltpu
from jax.experimental.pallas import tpu_sc as plsc

N = 262144
C = 64
GRID = 64
TABLE_ROWS = GRID * GRID

_NC = 2
_NS = 16
_NW = _NC * _NS
_L = 16

_QB = 128
_NQB = N // _QB
_CPW = _NQB // _NW
_PER_W = N // _NW
_PSTR = 129


def _body(table_hbm, x_hbm, out_hbm, x_v, idx_v, w_v, rows0, rows1,
          pack0, pack1, gsem0, gsem1, osem0, osem1):
    wid = lax.axis_index("s") * _NC + lax.axis_index("c")
    qb0 = wid * _CPW

    pltpu.sync_copy(x_hbm.at[pl.ds(qb0, _CPW)], x_v)

    lanes = lax.iota(jnp.int32, _L)
    zeros = jnp.zeros((_L,), jnp.int32)
    cap = jnp.full((_L,), GRID - 1, jnp.int32)

    @plsc.parallel_loop(0, _CPW, unroll=2)
    def _comp(ci):
        for j in range(_QB // _L):
            x0 = x_v[ci, 0, pl.ds(j * _L, _L)]
            x1 = x_v[ci, 1, pl.ds(j * _L, _L)]
            low0 = x0.astype(jnp.int32)
            low1 = x1.astype(jnp.int32)
            w0 = (low0 + 1).astype(jnp.float32) - x0
            w1 = (low1 + 1).astype(jnp.float32) - x1
            q0 = ci * _QB + j * _L
            idx_v[pl.ds(q0, _L)] = jnp.minimum(low0, cap) * GRID + jnp.minimum(low1, cap)
            w_v[pl.ds(q0, _L)] = w0 * w1

    rows = (rows0, rows1)
    pack = (pack0, pack1)
    gsem = (gsem0, gsem1)
    osem = (osem0, osem1)

    def fire_gather(ci, p):
        pltpu.async_copy(
            table_hbm.at[idx_v.at[pl.ds(ci * _QB, _QB)]], rows[p], gsem[p])

    cdims = []
    for cb16 in range(C // _L):
        cvec = cb16 * _L + lanes
        cdims.append((cvec >> 3, zeros, cvec & 7))

    fire_gather(0, 0)

    def pair_body(cp, carry):
        for par in (0, 1):
            ci = 2 * cp + par
            q = 1 - par

            @pl.when(ci >= 1)
            def _wait_out():
                pltpu.make_async_copy(
                    pack[q].at[:, :, :, pl.ds(0, _QB)],
                    out_hbm.at[:, pl.ds(0, 1)], osem[q]).wait()

            @pl.when(ci + 1 < _CPW)
            def _next_gather():
                fire_gather(ci + 1, q)

            pltpu.make_async_copy(
                table_hbm.at[pl.ds(0, _QB)], rows[par], gsem[par]).wait()

            woff = ci * _QB
            rows_p = rows[par]
            pack_p = pack[par]

            @plsc.parallel_loop(0, _QB, step=8, unroll=2)
            def _scale(b0):
                wvec = w_v[pl.ds(woff + b0, _L)]
                for k in range(8):
                    b = b0 + k
                    wq = jnp.full((_L,), wvec[k])
                    qvec = zeros + b
                    for cb16 in range(C // _L):
                        seg = rows_p[b, pl.ds(cb16 * _L, _L)]
                        d0, d1, d2 = cdims[cb16]
                        plsc.store_scatter(
                            pack_p, [d0, d1, d2, qvec], seg * wq)

            pltpu.async_copy(
                pack[par].at[:, :, :, pl.ds(0, _QB)],
                out_hbm.at[:, pl.ds(qb0 + ci, 1)], osem[par])
        return carry

    lax.fori_loop(0, _CPW // 2, pair_body, 0)

    pltpu.make_async_copy(
        pack1.at[:, :, :, pl.ds(0, _QB)],
        out_hbm.at[:, pl.ds(0, 1)], osem1).wait()


_kern = pl.kernel(
    _body,
    out_type=jax.ShapeDtypeStruct((C // 8, _NQB, 8, _QB), jnp.float32),
    mesh=plsc.VectorSubcoreMesh(core_axis_name="c", subcore_axis_name="s"),
    scratch_types=[
        pltpu.VMEM((_CPW, 2, _QB), jnp.float32),
        pltpu.VMEM((_PER_W,), jnp.int32),
        pltpu.VMEM((_PER_W + _L,), jnp.float32),
        pltpu.VMEM((_QB, C), jnp.float32),
        pltpu.VMEM((_QB, C), jnp.float32),
        pltpu.VMEM((C // 8, 1, 8, _PSTR), jnp.float32),
        pltpu.VMEM((C // 8, 1, 8, _PSTR), jnp.float32),
        pltpu.SemaphoreType.DMA,
        pltpu.SemaphoreType.DMA,
        pltpu.SemaphoreType.DMA,
        pltpu.SemaphoreType.DMA,
    ],
    compiler_params=pltpu.CompilerParams(
        needs_layout_passes=False, use_tc_tiling_on_sc=False
    ),
)


def kernel(image, x):
    table = image.reshape(TABLE_ROWS, C)
    xp = x.reshape(_NQB, _QB, 2).transpose(0, 2, 1)
    out_l = _kern(table, xp)
    return out_l.transpose(1, 3, 0, 2).reshape(N, C)
